# Initial kernel scaffold; baseline (speedup 1.0000x reference)
import jax
import jax.numpy as jnp
from jax.experimental import pallas as pl


def kernel(x, pos, triple_index, triple_attr, ln_gamma, ln_beta, W1, b1, W2, b2, W3, b3, Wout):
    raise NotImplementedError("write your pallas kernel here")


# speedup vs baseline: 1.3729x; 1.3729x over previous
import jax
import jax.numpy as jnp
from jax import lax
from jax.experimental import pallas as pl
from jax.experimental.pallas import tpu as pltpu
from jax.experimental.pallas import tpu_sc as plsc

F32 = jnp.float32
BF16 = jnp.bfloat16

_SD = 256
_NA = 160000
_NN = 10000
_IN = 515

_NB_A = 2000


def _prep_body(x_ref, pos_ref, gT_ref, bT_ref, w1_ref, b1_ref,
               p_ref, q_ref, s_ref, aux_ref):
    xb = x_ref[...]
    w1g = w1_ref[...] * gT_ref[...]
    xb16 = xb.astype(BF16)
    w1g16 = w1g.astype(BF16)
    p_ref[...] = jnp.dot(xb16, w1g16[0:256],
                         preferred_element_type=F32).astype(BF16)
    q_ref[...] = jnp.dot(xb16, w1g16[256:512],
                         preferred_element_type=F32).astype(BF16)
    s = jnp.sum(xb, axis=1, keepdims=True)
    ss = jnp.sum(xb * xb, axis=1, keepdims=True)
    posb = pos_ref[...][:, 0:3]
    s_ref[...] = jnp.concatenate(
        [s, ss, posb, jnp.zeros((_NB_A, 11), F32)], axis=1)

    @pl.when(pl.program_id(0) == 0)
    def _():
        u = jnp.sum(w1g, axis=0, keepdims=True)
        v = jnp.sum(w1_ref[...] * bT_ref[...], axis=0, keepdims=True) \
            + b1_ref[...]
        aux_ref[...] = jnp.concatenate(
            [u, v, w1g[512:515], jnp.zeros((3, 256), F32)], axis=0)


def _stage_a(x, pos8, gT, bT, W1, b1r):
    grid = (_NN // _NB_A,)
    return pl.pallas_call(
        _prep_body,
        grid=grid,
        in_specs=[
            pl.BlockSpec((_NB_A, _SD), lambda i: (i, 0)),
            pl.BlockSpec((_NB_A, 8), lambda i: (i, 0)),
            pl.BlockSpec((_IN, 1), lambda i: (0, 0)),
            pl.BlockSpec((_IN, 1), lambda i: (0, 0)),
            pl.BlockSpec((_IN, _SD), lambda i: (0, 0)),
            pl.BlockSpec((1, _SD), lambda i: (0, 0)),
        ],
        out_specs=[
            pl.BlockSpec((_NB_A, _SD), lambda i: (i, 0)),
            pl.BlockSpec((_NB_A, _SD), lambda i: (i, 0)),
            pl.BlockSpec((_NB_A, 16), lambda i: (i, 0)),
            pl.BlockSpec((8, _SD), lambda i: (0, 0)),
        ],
        out_shape=[
            jax.ShapeDtypeStruct((_NN, _SD), BF16),
            jax.ShapeDtypeStruct((_NN, _SD), BF16),
            jax.ShapeDtypeStruct((_NN, 16), F32),
            jax.ShapeDtypeStruct((8, _SD), F32),
        ],
    )(x, pos8, gT, bT, W1, b1r)


_GW = 128


def _gather_kernel(p_hbm, q_hbm, s_hbm, ii_hbm, jj_hbm, kk_hbm,
                   pi_hbm, qk_hbm, si_hbm, sj_hbm, sk_hbm):
    def body(ii_v, jj_v, kk_v, pi_v, qk_v, si_v, sj_v, sk_v):
        pltpu.sync_copy(p_hbm.at[ii_v.at[0]], pi_v)
        pltpu.sync_copy(q_hbm.at[kk_v.at[0]], qk_v)
        pltpu.sync_copy(s_hbm.at[ii_v.at[0]], si_v)
        pltpu.sync_copy(s_hbm.at[jj_v.at[0]], sj_v)
        pltpu.sync_copy(s_hbm.at[kk_v.at[0]], sk_v)

    pltpu.emit_pipeline(
        body,
        grid=(_NA // _GW,),
        in_specs=[pl.BlockSpec((1, _GW), lambda g: (0, g))] * 3,
        out_specs=[
            pl.BlockSpec((_GW, 128), lambda g: (g, 0)),
            pl.BlockSpec((_GW, 128), lambda g: (g, 0)),
            pl.BlockSpec((_GW, 16), lambda g: (g, 0)),
            pl.BlockSpec((_GW, 16), lambda g: (g, 0)),
            pl.BlockSpec((_GW, 16), lambda g: (g, 0)),
        ],
        core_axis_name=("core", "subcore"),
        dimension_semantics=(pltpu.PARALLEL,),
    )(ii_hbm, jj_hbm, kk_hbm, pi_hbm, qk_hbm, si_hbm, sj_hbm, sk_hbm)


def _stage_b(p_i32, q_i32, S, ii2, jj2, kk2):
    mesh = plsc.VectorSubcoreMesh(core_axis_name="core",
                                  subcore_axis_name="subcore")
    run = pl.kernel(
        _gather_kernel,
        compiler_params=pltpu.CompilerParams(use_tc_tiling_on_sc=False),
        out_type=[
            jax.ShapeDtypeStruct((_NA, 128), jnp.int32),
            jax.ShapeDtypeStruct((_NA, 128), jnp.int32),
            jax.ShapeDtypeStruct((_NA, 16), F32),
            jax.ShapeDtypeStruct((_NA, 16), F32),
            jax.ShapeDtypeStruct((_NA, 16), F32),
        ],
        mesh=mesh,
    )
    return run(p_i32, q_i32, S, ii2, jj2, kk2)


_BA = 1600


def _mlp_body(pi_ref, qk_ref, si_ref, sj_ref, sk_ref, attr_ref, aux_ref,
              w2_ref, w3_ref, b2_ref, b3_ref, msg_ref):
    si = si_ref[...]
    sj = sj_ref[...]
    sk = sk_ref[...]
    dxi = si[:, 2:3] - sj[:, 2:3]
    dyi = si[:, 3:4] - sj[:, 3:4]
    dzi = si[:, 4:5] - sj[:, 4:5]
    dxk = sk[:, 2:3] - sj[:, 2:3]
    dyk = sk[:, 3:4] - sj[:, 3:4]
    dzk = sk[:, 4:5] - sj[:, 4:5]
    dot = dxi * dxk + dyi * dyk + dzi * dzk
    n1 = jnp.maximum(dxi * dxi + dyi * dyi + dzi * dzi, 1e-12)
    n2 = jnp.maximum(dxk * dxk + dyk * dyk + dzk * dzk, 1e-12)
    cos = jnp.clip(dot * lax.rsqrt(n1 * n2), -1.0, 1.0)

    a0 = attr_ref[...][:, 0:1]
    a1 = attr_ref[...][:, 1:2]
    s1 = si[:, 0:1] + sk[:, 0:1] + a0 + a1 + cos
    s2 = si[:, 1:2] + sk[:, 1:2] + a0 * a0 + a1 * a1 + cos * cos
    inv_n = F32(1.0 / _IN)
    mu = s1 * inv_n
    var = s2 * inv_n - mu * mu
    rsd = lax.rsqrt(var + 1e-5)

    aux = aux_ref[...]
    u = aux[0:1, :]
    v = aux[1:2, :]
    w0 = aux[2:3, :]
    w1r = aux[3:4, :]
    w2r = aux[4:5, :]

    base = (pi_ref[...].astype(F32) + qk_ref[...].astype(F32)
            + a0 * w0 + a1 * w1r + cos * w2r)
    t = base * rsd - (mu * rsd) * u + v
    h1 = t * jax.nn.sigmoid(t)
    h2 = jnp.dot(h1.astype(BF16), w2_ref[...],
                 preferred_element_type=F32) + b2_ref[...]
    h2 = h2 * jax.nn.sigmoid(h2)
    msg_ref[...] = jnp.dot(h2.astype(BF16), w3_ref[...],
                           preferred_element_type=F32) + b3_ref[...]


def _stage_c(pi_bf, qk_bf, si_g, sj_g, sk_g, attr, aux, W2b, W3b, b2r, b3r):
    grid = (_NA // _BA,)
    return pl.pallas_call(
        _mlp_body,
        grid=grid,
        in_specs=[
            pl.BlockSpec((_BA, _SD), lambda i: (i, 0)),
            pl.BlockSpec((_BA, _SD), lambda i: (i, 0)),
            pl.BlockSpec((_BA, 16), lambda i: (i, 0)),
            pl.BlockSpec((_BA, 16), lambda i: (i, 0)),
            pl.BlockSpec((_BA, 16), lambda i: (i, 0)),
            pl.BlockSpec((_BA, 2), lambda i: (i, 0)),
            pl.BlockSpec((8, _SD), lambda i: (0, 0)),
            pl.BlockSpec((_SD, _SD), lambda i: (0, 0)),
            pl.BlockSpec((_SD, _SD), lambda i: (0, 0)),
            pl.BlockSpec((1, _SD), lambda i: (0, 0)),
            pl.BlockSpec((1, _SD), lambda i: (0, 0)),
        ],
        out_specs=pl.BlockSpec((_BA, _SD), lambda i: (i, 0)),
        out_shape=jax.ShapeDtypeStruct((_NA, _SD), F32),
    )(pi_bf, qk_bf, si_g, sj_g, sk_g, attr, aux, W2b, W3b, b2r, b3r)


_C2 = 80
_PER_SUB = _NA // 16
_NITER = _PER_SUB // _C2


def _scatter_kernel(msg_hbm, jj_hbm, zeros_hbm, ones_hbm,
                    accout_hbm, cntout_hbm,
                    mbuf, jbuf, ones_v, acc, cnt, insem):
    c = lax.axis_index("core")
    s = lax.axis_index("subcore")

    pltpu.sync_copy(zeros_hbm.at[pl.ds(0, 625)], acc.at[pl.ds(s * 625, 625)])

    @pl.when(c == 0)
    def _():
        pltpu.sync_copy(zeros_hbm.at[pl.ds(0, 626), pl.ds(0, 16)],
                        cnt.at[pl.ds(s * 626, 626)])

    pltpu.sync_copy(ones_hbm, ones_v)
    plsc.subcore_barrier()

    def start_in(g, b):
        base = s * _PER_SUB + g * _C2
        pltpu.async_copy(jj_hbm.at[0, pl.ds(base, _C2)], jbuf.at[b, 0], insem)

        @pl.when(c == 0)
        def _():
            pltpu.async_copy(msg_hbm.at[pl.ds(base, _C2), pl.ds(0, 128)],
                             mbuf.at[b], insem)

        @pl.when(c == 1)
        def _():
            pltpu.async_copy(msg_hbm.at[pl.ds(base, _C2), pl.ds(128, 128)],
                             mbuf.at[b], insem)

    def wait_in(b):
        pltpu.make_async_copy(jj_hbm.at[0, pl.ds(0, _C2)],
                              jbuf.at[b, 0], insem).wait()
        pltpu.make_async_copy(msg_hbm.at[pl.ds(0, _C2), pl.ds(0, 128)],
                              mbuf.at[b], insem).wait()

    start_in(0, 0)

    @pl.loop(0, _NITER)
    def _(g):
        b = lax.rem(g, 2)
        wait_in(b)

        @pl.when(g < _NITER - 1)
        def _():
            start_in(g + 1, 1 - b)

        pltpu.sync_copy(mbuf.at[b], acc.at[jbuf.at[b, 0]], add=True)

        @pl.when(c == 0)
        def _():
            pltpu.sync_copy(ones_v, cnt.at[jbuf.at[b, 0]], add=True)

    plsc.subcore_barrier()
    pltpu.sync_copy(acc.at[pl.ds(s * 625, 625)],
                    accout_hbm.at[c, pl.ds(s * 625, 625)])

    @pl.when(c == 0)
    def _():
        pltpu.sync_copy(cnt.at[pl.ds(s * 626, 626)],
                        cntout_hbm.at[pl.ds(s * 626, 626)])


def _stage_d(msg, jj2, zeros, ones):
    mesh = plsc.VectorSubcoreMesh(core_axis_name="core",
                                  subcore_axis_name="subcore")
    run = pl.kernel(
        _scatter_kernel,
        compiler_params=pltpu.CompilerParams(use_tc_tiling_on_sc=False),
        out_type=[
            jax.ShapeDtypeStruct((2, _NN, 128), F32),
            jax.ShapeDtypeStruct((10016, 16), F32),
        ],
        mesh=mesh,
        scratch_types=[
            pltpu.VMEM((2, _C2, 128), F32),
            pltpu.VMEM((2, 1, _C2), jnp.int32),
            pltpu.VMEM((_C2, 16), F32),
            pltpu.VMEM_SHARED((_NN, 128), F32),
            pltpu.VMEM_SHARED((10016, 16), F32),
            pltpu.SemaphoreType.DMA,
        ],
    )
    return run(msg, jj2, zeros, ones)


_NB_E = 1000


def _final_body(a0_ref, a1_ref, cnt_ref, wout_ref, o_ref):
    aggr = jnp.concatenate([a0_ref[...], a1_ref[...]], axis=1)
    inv = 1.0 / jnp.maximum(cnt_ref[...], 1.0)
    aggr = aggr * inv
    o_ref[...] = lax.dot_general(
        aggr, wout_ref[...], (((1,), (0,)), ((), ())),
        precision=lax.Precision.HIGHEST) * F32(1.0 / 16.0)


def _stage_e(acc0, acc1, cntcol, Wout):
    grid = (_NN // _NB_E,)
    return pl.pallas_call(
        _final_body,
        grid=grid,
        in_specs=[
            pl.BlockSpec((_NB_E, 128), lambda i: (i, 0)),
            pl.BlockSpec((_NB_E, 128), lambda i: (i, 0)),
            pl.BlockSpec((_NB_E, 1), lambda i: (i, 0)),
            pl.BlockSpec((_SD, _SD), lambda i: (0, 0)),
        ],
        out_specs=pl.BlockSpec((_NB_E, _SD), lambda i: (i, 0)),
        out_shape=jax.ShapeDtypeStruct((_NN, _SD), F32),
    )(acc0, acc1, cntcol, Wout)


def kernel(x, pos, triple_index, triple_attr, ln_gamma, ln_beta,
           W1, b1, W2, b2, W3, b3, Wout):
    x = x.astype(F32)
    ti = triple_index.astype(jnp.int32)
    ii2 = ti[0].reshape(1, _NA)
    jj2 = ti[1].reshape(1, _NA)
    kk2 = ti[2].reshape(1, _NA)

    pos8 = jnp.pad(pos.astype(F32), ((0, 0), (0, 5)))
    gT = ln_gamma.reshape(_IN, 1).astype(F32)
    bT = ln_beta.reshape(_IN, 1).astype(F32)
    b1r = b1.reshape(1, _SD)
    b2r = b2.reshape(1, _SD)
    b3r = b3.reshape(1, _SD)

    P, Q, S, aux = _stage_a(x, pos8, gT, bT, W1, b1r)

    p_i32 = lax.bitcast_convert_type(P.reshape(_NN, 128, 2), jnp.int32)
    q_i32 = lax.bitcast_convert_type(Q.reshape(_NN, 128, 2), jnp.int32)

    pi_g, qk_g, si_g, sj_g, sk_g = _stage_b(p_i32, q_i32, S, ii2, jj2, kk2)

    pi_bf = lax.bitcast_convert_type(pi_g, BF16).reshape(_NA, _SD)
    qk_bf = lax.bitcast_convert_type(qk_g, BF16).reshape(_NA, _SD)

    msg = _stage_c(pi_bf, qk_bf, si_g, sj_g, sk_g, triple_attr, aux,
                   W2.astype(BF16), W3.astype(BF16), b2r, b3r)

    zeros = jnp.zeros((640, 128), F32)
    ones = jnp.ones((_C2, 16), F32)
    accout, cntout = _stage_d(msg, jj2, zeros, ones)

    cntcol = cntout[:_NN, 0:1]
    out = _stage_e(accout[0], accout[1], cntcol, Wout)
    return out


# speedup vs baseline: 2.1926x; 1.5971x over previous
import jax
import jax.numpy as jnp
from jax import lax
from jax.experimental import pallas as pl
from jax.experimental.pallas import tpu as pltpu
from jax.experimental.pallas import tpu_sc as plsc

F32 = jnp.float32
BF16 = jnp.bfloat16

_SD = 256
_NA = 160000
_NN = 10000
_IN = 515

_NB_A = 2000


def _prep_body(x_ref, pos_ref, gT_ref, bT_ref, w1_ref, b1_ref,
               p_ref, q_ref, s_ref, aux_ref):
    xb = x_ref[...]
    w1g = w1_ref[...] * gT_ref[...]
    xb16 = xb.astype(BF16)
    w1g16 = w1g.astype(BF16)
    p_ref[...] = jnp.dot(xb16, w1g16[0:256],
                         preferred_element_type=F32).astype(BF16)
    q_ref[...] = jnp.dot(xb16, w1g16[256:512],
                         preferred_element_type=F32).astype(BF16)
    s = jnp.sum(xb, axis=1, keepdims=True)
    ss = jnp.sum(xb * xb, axis=1, keepdims=True)
    posb = pos_ref[...][:, 0:3]
    s_ref[...] = jnp.concatenate(
        [s, ss, posb, jnp.zeros((_NB_A, 11), F32)], axis=1)

    @pl.when(pl.program_id(0) == 0)
    def _():
        u = jnp.sum(w1g, axis=0, keepdims=True)
        v = jnp.sum(w1_ref[...] * bT_ref[...], axis=0, keepdims=True) \
            + b1_ref[...]
        aux_ref[...] = jnp.concatenate(
            [u, v, w1g[512:515], jnp.zeros((3, 256), F32)], axis=0)


def _stage_a(x, pos8, gT, bT, W1, b1r):
    grid = (_NN // _NB_A,)
    return pl.pallas_call(
        _prep_body,
        grid=grid,
        in_specs=[
            pl.BlockSpec((_NB_A, _SD), lambda i: (i, 0)),
            pl.BlockSpec((_NB_A, 8), lambda i: (i, 0)),
            pl.BlockSpec((_IN, 1), lambda i: (0, 0)),
            pl.BlockSpec((_IN, 1), lambda i: (0, 0)),
            pl.BlockSpec((_IN, _SD), lambda i: (0, 0)),
            pl.BlockSpec((1, _SD), lambda i: (0, 0)),
        ],
        out_specs=[
            pl.BlockSpec((_NB_A, _SD), lambda i: (i, 0)),
            pl.BlockSpec((_NB_A, _SD), lambda i: (i, 0)),
            pl.BlockSpec((_NB_A, 16), lambda i: (i, 0)),
            pl.BlockSpec((8, _SD), lambda i: (0, 0)),
        ],
        out_shape=[
            jax.ShapeDtypeStruct((_NN, _SD), BF16),
            jax.ShapeDtypeStruct((_NN, _SD), BF16),
            jax.ShapeDtypeStruct((_NN, 16), F32),
            jax.ShapeDtypeStruct((8, _SD), F32),
        ],
    )(x, pos8, gT, bT, W1, b1r)


_GW = 128


def _gather_kernel(p_hbm, q_hbm, s_hbm, ti_hbm,
                   pi_hbm, qk_hbm, si_hbm, sj_hbm, sk_hbm):
    def body(ii_v, jj_v, kk_v, pi_v, qk_v, si_v, sj_v, sk_v):
        pltpu.sync_copy(p_hbm.at[ii_v.at[0]], pi_v)
        pltpu.sync_copy(q_hbm.at[kk_v.at[0]], qk_v)
        pltpu.sync_copy(s_hbm.at[ii_v.at[0]], si_v)
        pltpu.sync_copy(s_hbm.at[jj_v.at[0]], sj_v)
        pltpu.sync_copy(s_hbm.at[kk_v.at[0]], sk_v)

    pltpu.emit_pipeline(
        body,
        grid=(_NA // _GW,),
        in_specs=[
            pl.BlockSpec((1, _GW), lambda g: (0, g)),
            pl.BlockSpec((1, _GW), lambda g: (1, g)),
            pl.BlockSpec((1, _GW), lambda g: (2, g)),
        ],
        out_specs=[
            pl.BlockSpec((_GW, _SD), lambda g: (g, 0)),
            pl.BlockSpec((_GW, _SD), lambda g: (g, 0)),
            pl.BlockSpec((_GW, 16), lambda g: (g, 0)),
            pl.BlockSpec((_GW, 16), lambda g: (g, 0)),
            pl.BlockSpec((_GW, 16), lambda g: (g, 0)),
        ],
        core_axis_name=("core", "subcore"),
        dimension_semantics=(pltpu.PARALLEL,),
    )(ti_hbm, ti_hbm, ti_hbm, pi_hbm, qk_hbm, si_hbm, sj_hbm, sk_hbm)


def _stage_b(P, Q, S, ti):
    mesh = plsc.VectorSubcoreMesh(core_axis_name="core",
                                  subcore_axis_name="subcore")
    run = pl.kernel(
        _gather_kernel,
        compiler_params=pltpu.CompilerParams(use_tc_tiling_on_sc=False),
        out_type=[
            jax.ShapeDtypeStruct((_NA, _SD), BF16),
            jax.ShapeDtypeStruct((_NA, _SD), BF16),
            jax.ShapeDtypeStruct((_NA, 16), F32),
            jax.ShapeDtypeStruct((_NA, 16), F32),
            jax.ShapeDtypeStruct((_NA, 16), F32),
        ],
        mesh=mesh,
    )
    return run(P, Q, S, ti)


_BA = 1600


def _mlp_body(pi_ref, qk_ref, si_ref, sj_ref, sk_ref, attr_ref, aux_ref,
              w2_ref, w3_ref, b2_ref, b3_ref, msg_ref):
    si = si_ref[...]
    sj = sj_ref[...]
    sk = sk_ref[...]
    dxi = si[:, 2:3] - sj[:, 2:3]
    dyi = si[:, 3:4] - sj[:, 3:4]
    dzi = si[:, 4:5] - sj[:, 4:5]
    dxk = sk[:, 2:3] - sj[:, 2:3]
    dyk = sk[:, 3:4] - sj[:, 3:4]
    dzk = sk[:, 4:5] - sj[:, 4:5]
    dot = dxi * dxk + dyi * dyk + dzi * dzk
    n1 = jnp.maximum(dxi * dxi + dyi * dyi + dzi * dzi, 1e-12)
    n2 = jnp.maximum(dxk * dxk + dyk * dyk + dzk * dzk, 1e-12)
    cos = jnp.clip(dot * lax.rsqrt(n1 * n2), -1.0, 1.0)

    a0 = attr_ref[...][:, 0:1]
    a1 = attr_ref[...][:, 1:2]
    s1 = si[:, 0:1] + sk[:, 0:1] + a0 + a1 + cos
    s2 = si[:, 1:2] + sk[:, 1:2] + a0 * a0 + a1 * a1 + cos * cos
    inv_n = F32(1.0 / _IN)
    mu = s1 * inv_n
    var = s2 * inv_n - mu * mu
    rsd = lax.rsqrt(var + 1e-5)

    aux = aux_ref[...]
    u = aux[0:1, :]
    v = aux[1:2, :]
    w0 = aux[2:3, :]
    w1r = aux[3:4, :]
    w2r = aux[4:5, :]

    base = (pi_ref[...].astype(F32) + qk_ref[...].astype(F32)
            + a0 * w0 + a1 * w1r + cos * w2r)
    t = base * rsd - (mu * rsd) * u + v
    h1 = t * jax.nn.sigmoid(t)
    h2 = jnp.dot(h1.astype(BF16), w2_ref[...],
                 preferred_element_type=F32) + b2_ref[...]
    h2 = h2 * jax.nn.sigmoid(h2)
    msg_ref[...] = jnp.dot(h2.astype(BF16), w3_ref[...],
                           preferred_element_type=F32) + b3_ref[...]


def _stage_c(pi_bf, qk_bf, si_g, sj_g, sk_g, attr, aux, W2b, W3b, b2r, b3r):
    grid = (_NA // _BA,)
    return pl.pallas_call(
        _mlp_body,
        grid=grid,
        in_specs=[
            pl.BlockSpec((_BA, _SD), lambda i: (i, 0)),
            pl.BlockSpec((_BA, _SD), lambda i: (i, 0)),
            pl.BlockSpec((_BA, 16), lambda i: (i, 0)),
            pl.BlockSpec((_BA, 16), lambda i: (i, 0)),
            pl.BlockSpec((_BA, 16), lambda i: (i, 0)),
            pl.BlockSpec((_BA, 2), lambda i: (i, 0)),
            pl.BlockSpec((8, _SD), lambda i: (0, 0)),
            pl.BlockSpec((_SD, _SD), lambda i: (0, 0)),
            pl.BlockSpec((_SD, _SD), lambda i: (0, 0)),
            pl.BlockSpec((1, _SD), lambda i: (0, 0)),
            pl.BlockSpec((1, _SD), lambda i: (0, 0)),
        ],
        out_specs=pl.BlockSpec((_BA, _SD), lambda i: (i, 0)),
        out_shape=jax.ShapeDtypeStruct((_NA, _SD), F32),
    )(pi_bf, qk_bf, si_g, sj_g, sk_g, attr, aux, W2b, W3b, b2r, b3r)


_C2 = 80
_PER_SUB = _NA // 16
_NITER = _PER_SUB // _C2


def _scatter_kernel(msg_hbm, jj_hbm, zeros_hbm, ones_hbm,
                    accout_hbm, cntout_hbm,
                    mbuf, jbuf, ones_v, acc, cnt, insem):
    c = lax.axis_index("core")
    s = lax.axis_index("subcore")

    pltpu.sync_copy(zeros_hbm.at[pl.ds(0, 625)], acc.at[pl.ds(s * 625, 625)])

    @pl.when(c == 0)
    def _():
        pltpu.sync_copy(zeros_hbm.at[pl.ds(0, 626), pl.ds(0, 16)],
                        cnt.at[pl.ds(s * 626, 626)])

    pltpu.sync_copy(ones_hbm, ones_v)
    plsc.subcore_barrier()

    def start_in(g, b):
        base = s * _PER_SUB + g * _C2
        pltpu.async_copy(jj_hbm.at[1, pl.ds(base, _C2)], jbuf.at[b, 0], insem)

        @pl.when(c == 0)
        def _():
            pltpu.async_copy(msg_hbm.at[pl.ds(base, _C2), pl.ds(0, 128)],
                             mbuf.at[b], insem)

        @pl.when(c == 1)
        def _():
            pltpu.async_copy(msg_hbm.at[pl.ds(base, _C2), pl.ds(128, 128)],
                             mbuf.at[b], insem)

    def wait_in(b):
        pltpu.make_async_copy(jj_hbm.at[1, pl.ds(0, _C2)],
                              jbuf.at[b, 0], insem).wait()
        pltpu.make_async_copy(msg_hbm.at[pl.ds(0, _C2), pl.ds(0, 128)],
                              mbuf.at[b], insem).wait()

    start_in(0, 0)

    @pl.loop(0, _NITER)
    def _(g):
        b = lax.rem(g, 2)
        wait_in(b)

        @pl.when(g < _NITER - 1)
        def _():
            start_in(g + 1, 1 - b)

        pltpu.sync_copy(mbuf.at[b], acc.at[jbuf.at[b, 0]], add=True)

        @pl.when(c == 0)
        def _():
            pltpu.sync_copy(ones_v, cnt.at[jbuf.at[b, 0]], add=True)

    plsc.subcore_barrier()
    pltpu.sync_copy(acc.at[pl.ds(s * 625, 625)],
                    accout_hbm.at[c, pl.ds(s * 625, 625)])

    @pl.when(c == 0)
    def _():
        pltpu.sync_copy(cnt.at[pl.ds(s * 626, 626)],
                        cntout_hbm.at[pl.ds(s * 626, 626)])


def _stage_d(msg, jj2, zeros, ones):
    mesh = plsc.VectorSubcoreMesh(core_axis_name="core",
                                  subcore_axis_name="subcore")
    run = pl.kernel(
        _scatter_kernel,
        compiler_params=pltpu.CompilerParams(use_tc_tiling_on_sc=False),
        out_type=[
            jax.ShapeDtypeStruct((2, _NN, 128), F32),
            jax.ShapeDtypeStruct((10016, 16), F32),
        ],
        mesh=mesh,
        scratch_types=[
            pltpu.VMEM((2, _C2, 128), F32),
            pltpu.VMEM((2, 1, _C2), jnp.int32),
            pltpu.VMEM((_C2, 16), F32),
            pltpu.VMEM_SHARED((_NN, 128), F32),
            pltpu.VMEM_SHARED((10016, 16), F32),
            pltpu.SemaphoreType.DMA,
        ],
    )
    return run(msg, jj2, zeros, ones)


_NB_E = 1000


def _final_body(a0_ref, a1_ref, cnt_ref, wout_ref, o_ref):
    aggr = jnp.concatenate([a0_ref[0], a1_ref[0]], axis=1)
    inv = 1.0 / jnp.maximum(cnt_ref[...][:, 0:1], 1.0)
    aggr = aggr * inv
    o_ref[...] = lax.dot_general(
        aggr, wout_ref[...], (((1,), (0,)), ((), ())),
        precision=lax.Precision.HIGHEST) * F32(1.0 / 16.0)


def _stage_e(accout, cntout, Wout):
    grid = (_NN // _NB_E,)
    return pl.pallas_call(
        _final_body,
        grid=grid,
        in_specs=[
            pl.BlockSpec((1, _NB_E, 128), lambda i: (0, i, 0)),
            pl.BlockSpec((1, _NB_E, 128), lambda i: (1, i, 0)),
            pl.BlockSpec((_NB_E, 16), lambda i: (i, 0)),
            pl.BlockSpec((_SD, _SD), lambda i: (0, 0)),
        ],
        out_specs=pl.BlockSpec((_NB_E, _SD), lambda i: (i, 0)),
        out_shape=jax.ShapeDtypeStruct((_NN, _SD), F32),
    )(accout, accout, cntout, Wout)


def kernel(x, pos, triple_index, triple_attr, ln_gamma, ln_beta,
           W1, b1, W2, b2, W3, b3, Wout):
    x = x.astype(F32)
    ti = triple_index.astype(jnp.int32)

    pos8 = jnp.pad(pos.astype(F32), ((0, 0), (0, 5)))
    gT = ln_gamma.reshape(_IN, 1).astype(F32)
    bT = ln_beta.reshape(_IN, 1).astype(F32)
    b1r = b1.reshape(1, _SD)
    b2r = b2.reshape(1, _SD)
    b3r = b3.reshape(1, _SD)

    P, Q, S, aux = _stage_a(x, pos8, gT, bT, W1, b1r)

    pi_bf, qk_bf, si_g, sj_g, sk_g = _stage_b(P, Q, S, ti)

    msg = _stage_c(pi_bf, qk_bf, si_g, sj_g, sk_g, triple_attr, aux,
                   W2.astype(BF16), W3.astype(BF16), b2r, b3r)

    zeros = jnp.zeros((640, 128), F32)
    ones = jnp.ones((_C2, 16), F32)
    accout, cntout = _stage_d(msg, ti, zeros, ones)

    out = _stage_e(accout, cntout, Wout)
    return out


# speedup vs baseline: 5.2099x; 2.3761x over previous
import jax
import jax.numpy as jnp
from jax import lax
from jax.experimental import pallas as pl
from jax.experimental.pallas import tpu as pltpu
from jax.experimental.pallas import tpu_sc as plsc

F32 = jnp.float32
BF16 = jnp.bfloat16
U32 = jnp.uint32
I32 = jnp.int32

_SD = 256
_NA = 160000
_NN = 10000
_IN = 515


def _pack_pair(lo, hi):
    lob = lax.bitcast_convert_type(lo.astype(BF16).astype(F32), U32)
    hib = lax.bitcast_convert_type(hi.astype(BF16).astype(F32), U32)
    w = lax.shift_right_logical(lob, U32(16)) | (hib & U32(0xFFFF0000))
    return lax.bitcast_convert_type(w, I32)


_NB_A = 2000


def _prep_body(x_ref, gT_ref, bT_ref, w1_ref, b1_ref,
               p_ref, q_ref, sss_ref, aux_ref):
    xb = x_ref[...]
    w1g = w1_ref[...] * gT_ref[...]
    xb16 = xb.astype(BF16)
    w1g16 = w1g.astype(BF16)
    P = jnp.dot(xb16, w1g16[0:256], preferred_element_type=F32)
    Q = jnp.dot(xb16, w1g16[256:512], preferred_element_type=F32)
    p_ref[...] = _pack_pair(P[:, :128], P[:, 128:])
    q_ref[...] = _pack_pair(Q[:, :128], Q[:, 128:])
    s = jnp.sum(xb, axis=1, keepdims=True)
    ss = jnp.sum(xb * xb, axis=1, keepdims=True)
    sss_ref[...] = jnp.transpose(_pack_pair(s, ss))[None]

    @pl.when(pl.program_id(0) == 0)
    def _():
        u = jnp.sum(w1g, axis=0, keepdims=True)
        v = jnp.sum(w1_ref[...] * bT_ref[...], axis=0, keepdims=True) \
            + b1_ref[...]
        aux_ref[...] = jnp.concatenate(
            [u, v, w1g[512:515], jnp.zeros((3, 256), F32)], axis=0)


def _stage_a(x, gT, bT, W1, b1r):
    grid = (_NN // _NB_A,)
    return pl.pallas_call(
        _prep_body,
        grid=grid,
        in_specs=[
            pl.BlockSpec((_NB_A, _SD), lambda i: (i, 0)),
            pl.BlockSpec((_IN, 1), lambda i: (0, 0)),
            pl.BlockSpec((_IN, 1), lambda i: (0, 0)),
            pl.BlockSpec((_IN, _SD), lambda i: (0, 0)),
            pl.BlockSpec((1, _SD), lambda i: (0, 0)),
        ],
        out_specs=[
            pl.BlockSpec((_NB_A, 128), lambda i: (i, 0)),
            pl.BlockSpec((_NB_A, 128), lambda i: (i, 0)),
            pl.BlockSpec((1, 1, _NB_A), lambda i: (i, 0, 0)),
            pl.BlockSpec((8, _SD), lambda i: (0, 0)),
        ],
        out_shape=[
            jax.ShapeDtypeStruct((_NN, 128), I32),
            jax.ShapeDtypeStruct((_NN, 128), I32),
            jax.ShapeDtypeStruct((_NN // _NB_A, 1, _NB_A), I32),
            jax.ShapeDtypeStruct((8, _SD), F32),
        ],
    )(x, gT, bT, W1, b1r)


_GW = 128


def _sc_rsqrt(x):
    xi = lax.bitcast_convert_type(x, I32)
    yi = I32(0x5F3759DF) - lax.shift_right_logical(xi, I32(1))
    y = lax.bitcast_convert_type(yi, F32)
    xh = x * F32(0.5)
    for _ in range(3):
        y = y * (F32(1.5) - xh * y * y)
    return y


def _upk_lo(w):
    return lax.bitcast_convert_type(lax.shift_left(w, I32(16)), F32)


def _upk_hi(w):
    return lax.bitcast_convert_type(w & I32(-65536), F32)


def _gather_kernel(p_hbm, sss_hbm, pxy_hbm, pz_hbm, attrT_hbm, ti_hbm,
                   pi_hbm, scal_hbm,
                   pxy_v, pz_v, sss_v,
                   ii_b, jj_b, kk_b, a0_b, a1_b):
    pltpu.sync_copy(pxy_hbm, pxy_v)
    pltpu.sync_copy(pz_hbm, pz_v)
    for r in range(_NN // 2000):
        pltpu.sync_copy(sss_hbm.at[r, 0], sss_v.at[pl.ds(r * 2000, 2000)])
    inv_n = F32(1.0 / _IN)

    def body(idxs, pi_v, scal_v):
        g = idxs[0]
        pltpu.sync_copy(ti_hbm.at[0, pl.ds(g, 1), :], ii_b)
        pltpu.sync_copy(ti_hbm.at[1, pl.ds(g, 1), :], jj_b)
        pltpu.sync_copy(ti_hbm.at[2, pl.ds(g, 1), :], kk_b)
        pltpu.sync_copy(attrT_hbm.at[0, pl.ds(g, 1), :], a0_b)
        pltpu.sync_copy(attrT_hbm.at[1, pl.ds(g, 1), :], a1_b)
        pltpu.sync_copy(p_hbm.at[ii_b.at[0]], pi_v)
        for o in range(_GW // 16):
            sl = pl.ds(o * 16, 16)
            ii16 = ii_b[0, sl]
            jj16 = jj_b[0, sl]
            kk16 = kk_b[0, sl]
            wpi = plsc.load_gather(pxy_v, [ii16])
            wpj = plsc.load_gather(pxy_v, [jj16])
            wpk = plsc.load_gather(pxy_v, [kk16])
            pxi = _upk_lo(wpi)
            pyi = _upk_hi(wpi)
            pxj = _upk_lo(wpj)
            pyj = _upk_hi(wpj)
            pxk = _upk_lo(wpk)
            pyk = _upk_hi(wpk)
            pzi = plsc.load_gather(pz_v, [ii16])
            pzj = plsc.load_gather(pz_v, [jj16])
            pzk = plsc.load_gather(pz_v, [kk16])
            wi = plsc.load_gather(sss_v, [ii16])
            wk = plsc.load_gather(sss_v, [kk16])
            s_pair = _upk_lo(wi) + _upk_lo(wk)
            ss_pair = _upk_hi(wi) + _upk_hi(wk)
            dxi = pxi - pxj
            dyi = pyi - pyj
            dzi = pzi - pzj
            dxk = pxk - pxj
            dyk = pyk - pyj
            dzk = pzk - pzj
            dot = dxi * dxk + dyi * dyk + dzi * dzk
            n1 = jnp.maximum(dxi * dxi + dyi * dyi + dzi * dzi, F32(1e-12))
            n2 = jnp.maximum(dxk * dxk + dyk * dyk + dzk * dzk, F32(1e-12))
            cosv = dot * _sc_rsqrt(n1 * n2)
            cosv = jnp.minimum(jnp.maximum(cosv, F32(-1.0)), F32(1.0))
            a0 = a0_b[0, sl]
            a1 = a1_b[0, sl]
            s1 = s_pair + a0 + a1 + cosv
            s2 = ss_pair + a0 * a0 + a1 * a1 + cosv * cosv
            mu = s1 * inv_n
            var = s2 * inv_n - mu * mu
            rsd = _sc_rsqrt(var + F32(1e-5))
            scal_v[0, sl] = rsd
            scal_v[1, sl] = mu * rsd
            scal_v[2, sl] = a0
            scal_v[3, sl] = a1
            scal_v[4, sl] = cosv

    pltpu.emit_pipeline(
        body,
        grid=(_NA // _GW,),
        in_specs=[],
        out_specs=[
            pl.BlockSpec((_GW, 128), lambda g: (g, 0)),
            pl.BlockSpec((8, _GW), lambda g: (0, g)),
        ],
        core_axis_name=("core", "subcore"),
        dimension_semantics=(pltpu.PARALLEL,),
        _explicit_indices=True,
    )(pi_hbm, scal_hbm)


def _gather_q_kernel(q_hbm, ti_hbm, zeros_hbm, ones_hbm,
                     qk_hbm, cnt_hbm, ones_v, cnt_sp):
    c = lax.axis_index("core")
    s = lax.axis_index("subcore")

    pltpu.sync_copy(ones_hbm, ones_v)

    @pl.when(s < 15)
    def _():
        pltpu.sync_copy(zeros_hbm.at[pl.ds(0, 632)],
                        cnt_sp.at[pl.ds(s * 632, 632)])

    @pl.when(s == 15)
    def _():
        pltpu.sync_copy(zeros_hbm.at[pl.ds(0, 536)],
                        cnt_sp.at[pl.ds(9480, 536)])

    plsc.subcore_barrier()

    def body(jj_v, kk_v, qk_v):
        pltpu.sync_copy(q_hbm.at[kk_v.at[0, 0]], qk_v)
        pltpu.sync_copy(ones_v, cnt_sp.at[jj_v.at[0, 0]], add=True)

    pltpu.emit_pipeline(
        body,
        grid=(_NA // _GW,),
        in_specs=[pl.BlockSpec((1, 1, _GW), lambda g: (1, g, 0),
                              pipeline_mode=pl.Buffered(buffer_count=2)),
                  pl.BlockSpec((1, 1, _GW), lambda g: (2, g, 0),
                              pipeline_mode=pl.Buffered(buffer_count=2))],
        out_specs=[pl.BlockSpec((_GW, 128), lambda g: (g, 0))],
        core_axis_name=("core", "subcore"),
        dimension_semantics=(pltpu.PARALLEL,),
    )(ti_hbm, ti_hbm, qk_hbm)

    plsc.subcore_barrier()

    @pl.when(s < 15)
    def _():
        pltpu.sync_copy(cnt_sp.at[pl.ds(s * 632, 632)],
                        cnt_hbm.at[c, pl.ds(s * 632, 632)])

    @pl.when(s == 15)
    def _():
        pltpu.sync_copy(cnt_sp.at[pl.ds(9480, 536)],
                        cnt_hbm.at[c, pl.ds(9480, 536)])


def _stage_b(Pp, Qp, sss, pxy, pz, attrT, ti, zeros, ones):
    mesh = plsc.VectorSubcoreMesh(core_axis_name="core",
                                  subcore_axis_name="subcore")
    run1 = pl.kernel(
        _gather_kernel,
        compiler_params=pltpu.CompilerParams(use_tc_tiling_on_sc=True,
                                             needs_layout_passes=False),
        out_type=[
            jax.ShapeDtypeStruct((_NA, 128), I32),
            jax.ShapeDtypeStruct((8, _NA), F32),
        ],
        mesh=mesh,
        scratch_types=[
            pltpu.VMEM((_NN,), I32),
            pltpu.VMEM((_NN,), F32),
            pltpu.VMEM((_NN,), I32),
            pltpu.VMEM((1, _GW), I32),
            pltpu.VMEM((1, _GW), I32),
            pltpu.VMEM((1, _GW), I32),
            pltpu.VMEM((1, _GW), F32),
            pltpu.VMEM((1, _GW), F32),
        ],
    )
    pi_g, scal = run1(Pp, sss, pxy, pz, attrT, ti)
    run2 = pl.kernel(
        _gather_q_kernel,
        compiler_params=pltpu.CompilerParams(use_tc_tiling_on_sc=True,
                                             needs_layout_passes=False),
        out_type=[jax.ShapeDtypeStruct((_NA, 128), I32),
                  jax.ShapeDtypeStruct((2, 10016, 128), F32)],
        mesh=mesh,
        scratch_types=[
            pltpu.VMEM((_GW, 128), F32),
            pltpu.VMEM_SHARED((10016, 128), F32),
        ],
    )
    qk_g, cnt = run2(Qp, ti, zeros, ones)
    return pi_g, qk_g, scal, cnt


_BA = 1280


def _mlp_body(pi_ref, qk_ref, scal_ref, aux_ref, w2_ref, w3_ref,
              b2_ref, b3_ref, msg_ref):
    sc = jnp.transpose(scal_ref[...])
    rsd = sc[:, 0:1]
    murs = sc[:, 1:2]
    a0 = sc[:, 2:3]
    a1 = sc[:, 3:4]
    cos = sc[:, 4:5]

    aux = aux_ref[...]
    u = aux[0:1, :]
    v = aux[1:2, :]
    w0 = aux[2:3, :]
    w1r = aux[3:4, :]
    w2r = aux[4:5, :]

    pw = pi_ref[...]
    qw = qk_ref[...]
    pe = _upk_lo(pw) + _upk_lo(qw)
    po = _upk_hi(pw) + _upk_hi(qw)
    be = pe + a0 * w0[:, :128] + a1 * w1r[:, :128] + cos * w2r[:, :128]
    bo = po + a0 * w0[:, 128:] + a1 * w1r[:, 128:] + cos * w2r[:, 128:]
    te = be * rsd - murs * u[:, :128] + v[:, :128]
    to = bo * rsd - murs * u[:, 128:] + v[:, 128:]
    h = jnp.concatenate([te, to], axis=1)
    h = h * jax.nn.sigmoid(h)
    h2 = jnp.dot(h.astype(BF16), w2_ref[...],
                 preferred_element_type=F32) + b2_ref[...]
    h2 = h2 * jax.nn.sigmoid(h2)
    msg_ref[...] = jnp.dot(h2.astype(BF16), w3_ref[...],
                           preferred_element_type=F32) + b3_ref[...]


def _stage_c(pi_g, qk_g, scal, aux, W2b, W3b, b2r, b3r):
    grid = (_NA // _BA,)
    return pl.pallas_call(
        _mlp_body,
        grid=grid,
        in_specs=[
            pl.BlockSpec((_BA, 128), lambda i: (i, 0)),
            pl.BlockSpec((_BA, 128), lambda i: (i, 0)),
            pl.BlockSpec((8, _BA), lambda i: (0, i)),
            pl.BlockSpec((8, _SD), lambda i: (0, 0)),
            pl.BlockSpec((_SD, _SD), lambda i: (0, 0)),
            pl.BlockSpec((_SD, _SD), lambda i: (0, 0)),
            pl.BlockSpec((1, _SD), lambda i: (0, 0)),
            pl.BlockSpec((1, _SD), lambda i: (0, 0)),
        ],
        out_specs=pl.BlockSpec((_BA, _SD), lambda i: (i, 0)),
        out_shape=jax.ShapeDtypeStruct((_NA, _SD), F32),
    )(pi_g, qk_g, scal, aux, W2b, W3b, b2r, b3r)


_C2 = 128
_NCH = _NA // _C2
_DITER = -(-_NCH // 16)


def _scatter_kernel(msg_hbm, ti_hbm, zeros_hbm,
                    accout_hbm, mbuf, jbuf, acc, insem):
    c = lax.axis_index("core")
    s = lax.axis_index("subcore")

    @pl.when(s < 15)
    def _():
        pltpu.sync_copy(zeros_hbm.at[pl.ds(0, 632)],
                        acc.at[pl.ds(s * 632, 632)])

    @pl.when(s == 15)
    def _():
        pltpu.sync_copy(zeros_hbm.at[pl.ds(0, 520)],
                        acc.at[pl.ds(9480, 520)])

    plsc.subcore_barrier()

    def start_in(chunk, b):
        base = chunk * _C2
        pltpu.async_copy(ti_hbm.at[1, pl.ds(chunk, 1), :],
                         jbuf.at[b], insem)

        @pl.when(c == 0)
        def _():
            pltpu.async_copy(msg_hbm.at[pl.ds(base, _C2), pl.ds(0, 128)],
                             mbuf.at[b], insem)

        @pl.when(c == 1)
        def _():
            pltpu.async_copy(msg_hbm.at[pl.ds(base, _C2), pl.ds(128, 128)],
                             mbuf.at[b], insem)

    def wait_in(b):
        pltpu.make_async_copy(ti_hbm.at[1, pl.ds(0, 1), :],
                              jbuf.at[b], insem).wait()
        pltpu.make_async_copy(msg_hbm.at[pl.ds(0, _C2), pl.ds(0, 128)],
                              mbuf.at[b], insem).wait()

    start_in(s, 0)

    @pl.loop(0, _DITER)
    def _(g):
        chunk = s + g * 16
        b = lax.rem(g, 2)

        @pl.when(chunk < _NCH)
        def _():
            wait_in(b)

            @pl.when(chunk + 16 < _NCH)
            def _():
                start_in(chunk + 16, 1 - b)

            pltpu.sync_copy(mbuf.at[b], acc.at[jbuf.at[b, 0]], add=True)

    plsc.subcore_barrier()

    @pl.when(s < 15)
    def _():
        pltpu.sync_copy(acc.at[pl.ds(s * 632, 632)],
                        accout_hbm.at[c, pl.ds(s * 632, 632)])

    @pl.when(s == 15)
    def _():
        pltpu.sync_copy(acc.at[pl.ds(9480, 520)],
                        accout_hbm.at[c, pl.ds(9480, 520)])


def _stage_d(msg, ti, zeros):
    mesh = plsc.VectorSubcoreMesh(core_axis_name="core",
                                  subcore_axis_name="subcore")
    run = pl.kernel(
        _scatter_kernel,
        compiler_params=pltpu.CompilerParams(use_tc_tiling_on_sc=True,
                                             needs_layout_passes=False),
        out_type=[
            jax.ShapeDtypeStruct((2, _NN, 128), F32),
        ],
        mesh=mesh,
        scratch_types=[
            pltpu.VMEM((2, _C2, 128), F32),
            pltpu.VMEM((2, 1, _C2), I32),
            pltpu.VMEM_SHARED((_NN, 128), F32),
            pltpu.SemaphoreType.DMA,
        ],
    )
    return run(msg, ti, zeros)


_NB_E = 1000


def _final_body(a0_ref, a1_ref, c0_ref, c1_ref, wout_ref, o_ref):
    aggr = jnp.concatenate([a0_ref[0], a1_ref[0]], axis=1)
    cnt = c0_ref[0][:, 0:1] + c1_ref[0][:, 0:1]
    inv = 1.0 / jnp.maximum(cnt, 1.0)
    aggr = aggr * inv
    o_ref[...] = lax.dot_general(
        aggr, wout_ref[...], (((1,), (0,)), ((), ())),
        precision=lax.Precision.HIGHEST) * F32(1.0 / 16.0)


def _stage_e(accout, cnt, Wout):
    grid = (_NN // _NB_E,)
    return pl.pallas_call(
        _final_body,
        grid=grid,
        in_specs=[
            pl.BlockSpec((1, _NB_E, 128), lambda i: (0, i, 0)),
            pl.BlockSpec((1, _NB_E, 128), lambda i: (1, i, 0)),
            pl.BlockSpec((1, _NB_E, 128), lambda i: (0, i, 0)),
            pl.BlockSpec((1, _NB_E, 128), lambda i: (1, i, 0)),
            pl.BlockSpec((_SD, _SD), lambda i: (0, 0)),
        ],
        out_specs=pl.BlockSpec((_NB_E, _SD), lambda i: (i, 0)),
        out_shape=jax.ShapeDtypeStruct((_NN, _SD), F32),
    )(accout, accout, cnt, cnt, Wout)


def kernel(x, pos, triple_index, triple_attr, ln_gamma, ln_beta,
           W1, b1, W2, b2, W3, b3, Wout):
    x = x.astype(F32)
    ti = triple_index.astype(I32)
    pos32 = pos.astype(F32)
    pxb = lax.bitcast_convert_type(pos32[:, 0].astype(BF16).astype(F32), U32)
    pyb = lax.bitcast_convert_type(pos32[:, 1].astype(BF16).astype(F32), U32)
    pxy = lax.bitcast_convert_type(
        lax.shift_right_logical(pxb, U32(16)) | (pyb & U32(0xFFFF0000)), I32)
    pz = pos32[:, 2]
    attrT = triple_attr.astype(F32).T.reshape(2, _NA // _GW, _GW)
    ti3 = ti.reshape(3, _NA // _GW, _GW)

    gT = ln_gamma.reshape(_IN, 1).astype(F32)
    bT = ln_beta.reshape(_IN, 1).astype(F32)
    b1r = b1.reshape(1, _SD)
    b2r = b2.reshape(1, _SD)
    b3r = b3.reshape(1, _SD)

    Pp, Qp, sss, aux = _stage_a(x, gT, bT, W1, b1r)

    zeros = jnp.zeros((640, 128), F32)
    ones = jnp.ones((_GW, 128), F32)
    pi_g, qk_g, scal, cnt = _stage_b(Pp, Qp, sss, pxy, pz, attrT, ti3,
                                     zeros, ones)

    msg = _stage_c(pi_g, qk_g, scal, aux,
                   W2.astype(BF16), W3.astype(BF16), b2r, b3r)

    accout = _stage_d(msg, ti3, zeros)[0]

    out = _stage_e(accout, cnt, Wout)
    return out
